# baseline (speedup 1.0000x reference)
import functools

import jax
import jax.numpy as jnp
from jax.experimental import pallas as pl
from jax.experimental.pallas import tpu as pltpu


def _softplus(x):
    return jnp.maximum(x, 0.0) + jnp.log1p(jnp.exp(-jnp.abs(x)))


def _sigmoid(x):
    return 0.5 * (jnp.tanh(0.5 * x) + 1.0)


def _embed_proj_kernel(x_ref, w_ref, b_ref, wn_ref, o_ref, p_ref):
    x = (
        jnp.dot(x_ref[...], w_ref[...], preferred_element_type=jnp.float32)
        + b_ref[...]
    )
    o_ref[...] = x
    p_ref[...] = jnp.dot(x, wn_ref[...], preferred_element_type=jnp.float32)


def _conv_body(x_ref, p3_ref, idx_ref, nb_ref, ws_ref, wb_ref,
               b_ref, bn2s_ref, bn2b_ref, g2_ref, *, m):
    T, A = x_ref.shape

    U = 4
    E = U * m

    def gather_rows(tt, carry):
        base = tt * E
        idxs = [idx_ref[(tt * U) + u, mi]
                for u in range(U) for mi in range(m)]
        rows = [p3_ref[i] for i in idxs]
        for k in range(E // 8):
            dst = pl.multiple_of(base + k * 8, 8)
            g2_ref[pl.ds(dst, 8), :] = jnp.concatenate(
                rows[k * 8:(k + 1) * 8], axis=0)
        return carry

    jax.lax.fori_loop(0, T // U, gather_rows, 0)

    x = x_ref[...]
    zs = jnp.dot(x, ws_ref[...],
                 preferred_element_type=jnp.float32) + b_ref[...]
    zb = jnp.dot(nb_ref[...], wb_ref[...],
                 preferred_element_type=jnp.float32)
    zf = zb[:, :A] + g2_ref[:, :A]
    zc = zb[:, A:] + g2_ref[:, A:]
    zf = zf.reshape(T, m, A) + zs[:, None, :A]
    zc = zc.reshape(T, m, A) + zs[:, None, A:]
    gate = _sigmoid(zf) * _softplus(zc)
    summed = jnp.sum(gate, axis=1)
    return _softplus(x + summed * bn2s_ref[...] + bn2b_ref[...])


def _conv_proj_kernel(x_ref, p3_ref, idx_ref, nb_ref, ws_ref,
                      wb_ref, b_ref, bn2s_ref, bn2b_ref, wn_ref,
                      o_ref, p_ref, g2_ref, *, m):
    y = _conv_body(x_ref, p3_ref, idx_ref, nb_ref, ws_ref, wb_ref,
                   b_ref, bn2s_ref, bn2b_ref, g2_ref, m=m)
    o_ref[...] = y
    p_ref[...] = jnp.dot(y, wn_ref[...], preferred_element_type=jnp.float32)


def _conv_last_kernel(x_ref, p3_ref, idx_ref, nb_ref, ws_ref,
                      wb_ref, b_ref, bn2s_ref, bn2b_ref, o_ref,
                      g2_ref, *, m):
    o_ref[...] = _conv_body(x_ref, p3_ref, idx_ref, nb_ref, ws_ref,
                            wb_ref, b_ref, bn2s_ref, bn2b_ref,
                            g2_ref, m=m)


def _head_kernel(p_ref, x_ref, wc_ref, bc_ref, wo_ref, bo_ref, o_ref):
    c = jnp.dot(p_ref[...], x_ref[...], preferred_element_type=jnp.float32)
    h = _softplus(c)
    h = jnp.dot(h, wc_ref[...], preferred_element_type=jnp.float32) + bc_ref[...]
    h = _softplus(h)
    o_ref[...] = (
        jnp.dot(h, wo_ref[...], preferred_element_type=jnp.float32) + bo_ref[...]
    )


def kernel(atom_fea, nbr_fea, nbr_fea_idx, pool_mat, emb_w, emb_b, fc_w, fc_b, out_w, out_b, conv0_wsf, conv0_wsc, conv0_wnf, conv0_wnc, conv0_wbf, conv0_wbc, conv0_bf, conv0_bc, conv0_bn2_s, conv0_bn2_b, conv1_wsf, conv1_wsc, conv1_wnf, conv1_wnc, conv1_wbf, conv1_wbc, conv1_bf, conv1_bc, conv1_bn2_s, conv1_bn2_b, conv2_wsf, conv2_wsc, conv2_wnf, conv2_wnc, conv2_wbf, conv2_wbc, conv2_bf, conv2_bc, conv2_bn2_s, conv2_bn2_b):
    N, M = nbr_fea_idx.shape
    B = nbr_fea.shape[2]
    F = atom_fea.shape[1]
    A = emb_w.shape[1]
    N0 = pool_mat.shape[0]

    T = 512 if N % 512 == 0 else N
    G = N // T

    cparams = pltpu.CompilerParams(dimension_semantics=("parallel",))

    convs = [
        (conv0_wsf, conv0_wsc, conv0_wnf, conv0_wnc, conv0_wbf, conv0_wbc,
         conv0_bf, conv0_bc, conv0_bn2_s, conv0_bn2_b),
        (conv1_wsf, conv1_wsc, conv1_wnf, conv1_wnc, conv1_wbf, conv1_wbc,
         conv1_bf, conv1_bc, conv1_bn2_s, conv1_bn2_b),
        (conv2_wsf, conv2_wsc, conv2_wnf, conv2_wnc, conv2_wbf, conv2_wbc,
         conv2_bf, conv2_bc, conv2_bn2_s, conv2_bn2_b),
    ]
    ws = [jnp.concatenate([c[0], c[1]], axis=1) for c in convs]
    wn = [jnp.concatenate([c[2], c[3]], axis=1) for c in convs]
    wb = [jnp.concatenate([c[4], c[5]], axis=1) for c in convs]
    bias = [jnp.concatenate([c[6], c[7]], axis=1) for c in convs]
    bn2s = [c[8] for c in convs]
    bn2b = [c[9] for c in convs]

    nb_flat = nbr_fea.reshape(N * M, B)

    const = lambda shape: pl.BlockSpec(shape, lambda i: (0, 0))

    x, p = pl.pallas_call(
        _embed_proj_kernel,
        out_shape=(jax.ShapeDtypeStruct((N, A), jnp.float32),
                   jax.ShapeDtypeStruct((N, 2 * A), jnp.float32)),
        grid=(G,),
        in_specs=[pl.BlockSpec((T, F), lambda i: (i, 0)),
                  const((F, A)), const((1, A)), const((A, 2 * A))],
        out_specs=(pl.BlockSpec((T, A), lambda i: (i, 0)),
                   pl.BlockSpec((T, 2 * A), lambda i: (i, 0))),
        compiler_params=cparams,
    )(atom_fea, emb_w, emb_b, wn[0])

    conv_in_specs = [
        pl.BlockSpec((T, A), lambda i: (i, 0)),
        pl.BlockSpec((N, 1, 2 * A), lambda i: (0, 0, 0)),
        pl.BlockSpec((T, M), lambda i: (i, 0),
                     memory_space=pltpu.MemorySpace.SMEM),
        pl.BlockSpec((T * M, B), lambda i: (i, 0)),
        const((A, 2 * A)), const((B, 2 * A)), const((1, 2 * A)),
        const((1, A)), const((1, A)),
    ]
    scratch = [pltpu.VMEM((T * M, 2 * A), jnp.float32)]

    for layer in (0, 1):
        x, p = pl.pallas_call(
            functools.partial(_conv_proj_kernel, m=M),
            out_shape=(jax.ShapeDtypeStruct((N, A), jnp.float32),
                       jax.ShapeDtypeStruct((N, 2 * A), jnp.float32)),
            grid=(G,),
            in_specs=conv_in_specs + [const((A, 2 * A))],
            out_specs=(pl.BlockSpec((T, A), lambda i: (i, 0)),
                       pl.BlockSpec((T, 2 * A), lambda i: (i, 0))),
            scratch_shapes=scratch,
            compiler_params=cparams,
        )(x, p.reshape(N, 1, 2 * A), nbr_fea_idx, nb_flat,
          ws[layer], wb[layer], bias[layer], bn2s[layer], bn2b[layer],
          wn[layer + 1])

    x = pl.pallas_call(
        functools.partial(_conv_last_kernel, m=M),
        out_shape=jax.ShapeDtypeStruct((N, A), jnp.float32),
        grid=(G,),
        in_specs=conv_in_specs,
        out_specs=pl.BlockSpec((T, A), lambda i: (i, 0)),
        scratch_shapes=scratch,
        compiler_params=cparams,
    )(x, p.reshape(N, 1, 2 * A), nbr_fea_idx, nb_flat,
      ws[2], wb[2], bias[2], bn2s[2], bn2b[2])

    vspec = pl.BlockSpec(memory_space=pltpu.MemorySpace.VMEM)
    out = pl.pallas_call(
        _head_kernel,
        out_shape=jax.ShapeDtypeStruct((N0, 1), jnp.float32),
        in_specs=[vspec] * 6,
        out_specs=vspec,
    )(pool_mat, x, fc_w, fc_b, out_w, out_b)
    return out


# speedup vs baseline: 1.0556x; 1.0556x over previous
import functools

import jax
import jax.numpy as jnp
from jax.experimental import pallas as pl
from jax.experimental.pallas import tpu as pltpu


def _softplus(x):
    return jnp.maximum(x, 0.0) + jnp.log1p(jnp.exp(-jnp.abs(x)))


def _sigmoid(x):
    return 0.5 * (jnp.tanh(0.5 * x) + 1.0)


def _embed_proj_kernel(x_ref, w_ref, b_ref, wn_ref, o_ref, pf_ref, pc_ref):
    x = (
        jnp.dot(x_ref[...], w_ref[...], preferred_element_type=jnp.float32)
        + b_ref[...]
    )
    o_ref[...] = x
    A = x.shape[1]
    p = jnp.dot(x, wn_ref[...], preferred_element_type=jnp.float32)
    pf_ref[...] = p[:, :A]
    pc_ref[...] = p[:, A:]


def _conv_body(x_ref, pf3_ref, pc3_ref, idx_ref, nb_ref, ws_ref, wb_ref,
               b_ref, bn2s_ref, bn2b_ref, gf2_ref, gc2_ref, *, m):
    T, A = x_ref.shape

    U = 4
    E = U * m

    def gather_rows(tt, carry):
        base = tt * E
        idxs = [idx_ref[(tt * U) + u, mi]
                for u in range(U) for mi in range(m)]
        rowfs = [pf3_ref[i] for i in idxs]
        rowcs = [pc3_ref[i] for i in idxs]
        for k in range(E // 8):
            dst = pl.multiple_of(base + k * 8, 8)
            gf2_ref[pl.ds(dst, 8), :] = jnp.concatenate(
                rowfs[k * 8:(k + 1) * 8], axis=0)
            gc2_ref[pl.ds(dst, 8), :] = jnp.concatenate(
                rowcs[k * 8:(k + 1) * 8], axis=0)
        return carry

    jax.lax.fori_loop(0, T // U, gather_rows, 0)

    x = x_ref[...]
    zs = jnp.dot(x, ws_ref[...],
                 preferred_element_type=jnp.float32) + b_ref[...]
    zb = jnp.dot(nb_ref[...], wb_ref[...],
                 preferred_element_type=jnp.float32)
    zf = zb[:, :A] + gf2_ref[...]
    zc = zb[:, A:] + gc2_ref[...]
    zf = zf.reshape(T, m, A) + zs[:, None, :A]
    zc = zc.reshape(T, m, A) + zs[:, None, A:]
    gate = _sigmoid(zf) * _softplus(zc)
    summed = jnp.sum(gate, axis=1)
    return _softplus(x + summed * bn2s_ref[...] + bn2b_ref[...])


def _conv_proj_kernel(x_ref, pf3_ref, pc3_ref, idx_ref, nb_ref, ws_ref,
                      wb_ref, b_ref, bn2s_ref, bn2b_ref, wn_ref,
                      o_ref, pf_ref, pc_ref, gf2_ref, gc2_ref, *, m):
    y = _conv_body(x_ref, pf3_ref, pc3_ref, idx_ref, nb_ref, ws_ref, wb_ref,
                   b_ref, bn2s_ref, bn2b_ref, gf2_ref, gc2_ref, m=m)
    o_ref[...] = y
    A = y.shape[1]
    p = jnp.dot(y, wn_ref[...], preferred_element_type=jnp.float32)
    pf_ref[...] = p[:, :A]
    pc_ref[...] = p[:, A:]


def _conv_last_kernel(x_ref, pf3_ref, pc3_ref, idx_ref, nb_ref, ws_ref,
                      wb_ref, b_ref, bn2s_ref, bn2b_ref, o_ref,
                      gf2_ref, gc2_ref, *, m):
    o_ref[...] = _conv_body(x_ref, pf3_ref, pc3_ref, idx_ref, nb_ref, ws_ref,
                            wb_ref, b_ref, bn2s_ref, bn2b_ref,
                            gf2_ref, gc2_ref, m=m)


def _head_kernel(p_ref, x_ref, wc_ref, bc_ref, wo_ref, bo_ref, o_ref):
    c = jnp.dot(p_ref[...], x_ref[...], preferred_element_type=jnp.float32)
    h = _softplus(c)
    h = jnp.dot(h, wc_ref[...], preferred_element_type=jnp.float32) + bc_ref[...]
    h = _softplus(h)
    o_ref[...] = (
        jnp.dot(h, wo_ref[...], preferred_element_type=jnp.float32) + bo_ref[...]
    )


def kernel(atom_fea, nbr_fea, nbr_fea_idx, pool_mat, emb_w, emb_b, fc_w, fc_b, out_w, out_b, conv0_wsf, conv0_wsc, conv0_wnf, conv0_wnc, conv0_wbf, conv0_wbc, conv0_bf, conv0_bc, conv0_bn2_s, conv0_bn2_b, conv1_wsf, conv1_wsc, conv1_wnf, conv1_wnc, conv1_wbf, conv1_wbc, conv1_bf, conv1_bc, conv1_bn2_s, conv1_bn2_b, conv2_wsf, conv2_wsc, conv2_wnf, conv2_wnc, conv2_wbf, conv2_wbc, conv2_bf, conv2_bc, conv2_bn2_s, conv2_bn2_b):
    N, M = nbr_fea_idx.shape
    B = nbr_fea.shape[2]
    F = atom_fea.shape[1]
    A = emb_w.shape[1]
    N0 = pool_mat.shape[0]

    T = 512 if N % 512 == 0 else N
    G = N // T

    cparams = pltpu.CompilerParams(dimension_semantics=("parallel",))

    convs = [
        (conv0_wsf, conv0_wsc, conv0_wnf, conv0_wnc, conv0_wbf, conv0_wbc,
         conv0_bf, conv0_bc, conv0_bn2_s, conv0_bn2_b),
        (conv1_wsf, conv1_wsc, conv1_wnf, conv1_wnc, conv1_wbf, conv1_wbc,
         conv1_bf, conv1_bc, conv1_bn2_s, conv1_bn2_b),
        (conv2_wsf, conv2_wsc, conv2_wnf, conv2_wnc, conv2_wbf, conv2_wbc,
         conv2_bf, conv2_bc, conv2_bn2_s, conv2_bn2_b),
    ]
    ws = [jnp.concatenate([c[0], c[1]], axis=1) for c in convs]
    wn = [jnp.concatenate([c[2], c[3]], axis=1) for c in convs]
    wb = [jnp.concatenate([c[4], c[5]], axis=1) for c in convs]
    bias = [jnp.concatenate([c[6], c[7]], axis=1) for c in convs]
    bn2s = [c[8] for c in convs]
    bn2b = [c[9] for c in convs]

    nb_flat = nbr_fea.reshape(N * M, B)

    const = lambda shape: pl.BlockSpec(shape, lambda i: (0, 0))

    x, pf, pc = pl.pallas_call(
        _embed_proj_kernel,
        out_shape=(jax.ShapeDtypeStruct((N, A), jnp.float32),
                   jax.ShapeDtypeStruct((N, A), jnp.float32),
                   jax.ShapeDtypeStruct((N, A), jnp.float32)),
        grid=(G,),
        in_specs=[pl.BlockSpec((T, F), lambda i: (i, 0)),
                  const((F, A)), const((1, A)), const((A, 2 * A))],
        out_specs=(pl.BlockSpec((T, A), lambda i: (i, 0)),
                   pl.BlockSpec((T, A), lambda i: (i, 0)),
                   pl.BlockSpec((T, A), lambda i: (i, 0))),
        compiler_params=cparams,
    )(atom_fea, emb_w, emb_b, wn[0])

    conv_in_specs = [
        pl.BlockSpec((T, A), lambda i: (i, 0)),
        pl.BlockSpec((N, 1, A), lambda i: (0, 0, 0)),
        pl.BlockSpec((N, 1, A), lambda i: (0, 0, 0)),
        pl.BlockSpec((T, M), lambda i: (i, 0),
                     memory_space=pltpu.MemorySpace.SMEM),
        pl.BlockSpec((T * M, B), lambda i: (i, 0)),
        const((A, 2 * A)), const((B, 2 * A)), const((1, 2 * A)),
        const((1, A)), const((1, A)),
    ]
    scratch = [pltpu.VMEM((T * M, A), jnp.float32),
               pltpu.VMEM((T * M, A), jnp.float32)]

    for layer in (0, 1):
        x, pf, pc = pl.pallas_call(
            functools.partial(_conv_proj_kernel, m=M),
            out_shape=(jax.ShapeDtypeStruct((N, A), jnp.float32),
                       jax.ShapeDtypeStruct((N, A), jnp.float32),
                       jax.ShapeDtypeStruct((N, A), jnp.float32)),
            grid=(G,),
            in_specs=conv_in_specs + [const((A, 2 * A))],
            out_specs=(pl.BlockSpec((T, A), lambda i: (i, 0)),
                       pl.BlockSpec((T, A), lambda i: (i, 0)),
                       pl.BlockSpec((T, A), lambda i: (i, 0))),
            scratch_shapes=scratch,
            compiler_params=cparams,
        )(x, pf.reshape(N, 1, A), pc.reshape(N, 1, A), nbr_fea_idx, nb_flat,
          ws[layer], wb[layer], bias[layer], bn2s[layer], bn2b[layer],
          wn[layer + 1])

    x = pl.pallas_call(
        functools.partial(_conv_last_kernel, m=M),
        out_shape=jax.ShapeDtypeStruct((N, A), jnp.float32),
        grid=(G,),
        in_specs=conv_in_specs,
        out_specs=pl.BlockSpec((T, A), lambda i: (i, 0)),
        scratch_shapes=scratch,
        compiler_params=cparams,
    )(x, pf.reshape(N, 1, A), pc.reshape(N, 1, A), nbr_fea_idx, nb_flat,
      ws[2], wb[2], bias[2], bn2s[2], bn2b[2])

    vspec = pl.BlockSpec(memory_space=pltpu.MemorySpace.VMEM)
    out = pl.pallas_call(
        _head_kernel,
        out_shape=jax.ShapeDtypeStruct((N0, 1), jnp.float32),
        in_specs=[vspec] * 6,
        out_specs=vspec,
    )(pool_mat, x, fc_w, fc_b, out_w, out_b)
    return out


# speedup vs baseline: 1.0910x; 1.0335x over previous
import functools

import jax
import jax.numpy as jnp
from jax.experimental import pallas as pl
from jax.experimental.pallas import tpu as pltpu


def _softplus(x):
    return jnp.maximum(x, 0.0) + jnp.log1p(jnp.exp(-jnp.abs(x)))


def _sigmoid(x):
    return 0.5 * (jnp.tanh(0.5 * x) + 1.0)


def _embed_proj_kernel(x_ref, w_ref, b_ref, wn_ref, o_ref, pf_ref, pc_ref):
    x = (
        jnp.dot(x_ref[...], w_ref[...], preferred_element_type=jnp.float32)
        + b_ref[...]
    )
    o_ref[...] = x
    A = x.shape[1]
    p = jnp.dot(x, wn_ref[...], preferred_element_type=jnp.float32)
    pf_ref[...] = p[:, :A]
    pc_ref[...] = p[:, A:]


def _conv_body(x_ref, pf3_ref, pc3_ref, idx_ref, nb_ref, ws_ref, wb_ref,
               b_ref, bn2s_ref, bn2b_ref, gf2_ref, gc2_ref, *, m):
    T, A = x_ref.shape

    U = 8
    E = U * m

    def gather_rows(tt, carry):
        base = tt * E
        idxs = [idx_ref[(tt * U) + u, mi]
                for u in range(U) for mi in range(m)]
        rowfs = [pf3_ref[i] for i in idxs]
        rowcs = [pc3_ref[i] for i in idxs]
        for k in range(E // 8):
            dst = pl.multiple_of(base + k * 8, 8)
            gf2_ref[pl.ds(dst, 8), :] = jnp.concatenate(
                rowfs[k * 8:(k + 1) * 8], axis=0)
            gc2_ref[pl.ds(dst, 8), :] = jnp.concatenate(
                rowcs[k * 8:(k + 1) * 8], axis=0)
        return carry

    jax.lax.fori_loop(0, T // U, gather_rows, 0)

    x = x_ref[...]
    zs = jnp.dot(x, ws_ref[...],
                 preferred_element_type=jnp.float32) + b_ref[...]
    zb = jnp.dot(nb_ref[...], wb_ref[...],
                 preferred_element_type=jnp.float32)
    zf = zb[:, :A] + gf2_ref[...]
    zc = zb[:, A:] + gc2_ref[...]
    zf = zf.reshape(T, m, A) + zs[:, None, :A]
    zc = zc.reshape(T, m, A) + zs[:, None, A:]
    gate = _sigmoid(zf) * _softplus(zc)
    summed = jnp.sum(gate, axis=1)
    return _softplus(x + summed * bn2s_ref[...] + bn2b_ref[...])


def _conv_proj_kernel(x_ref, pf3_ref, pc3_ref, idx_ref, nb_ref, ws_ref,
                      wb_ref, b_ref, bn2s_ref, bn2b_ref, wn_ref,
                      o_ref, pf_ref, pc_ref, gf2_ref, gc2_ref, *, m):
    y = _conv_body(x_ref, pf3_ref, pc3_ref, idx_ref, nb_ref, ws_ref, wb_ref,
                   b_ref, bn2s_ref, bn2b_ref, gf2_ref, gc2_ref, m=m)
    o_ref[...] = y
    A = y.shape[1]
    p = jnp.dot(y, wn_ref[...], preferred_element_type=jnp.float32)
    pf_ref[...] = p[:, :A]
    pc_ref[...] = p[:, A:]


def _conv_last_kernel(x_ref, pf3_ref, pc3_ref, idx_ref, nb_ref, ws_ref,
                      wb_ref, b_ref, bn2s_ref, bn2b_ref, o_ref,
                      gf2_ref, gc2_ref, *, m):
    o_ref[...] = _conv_body(x_ref, pf3_ref, pc3_ref, idx_ref, nb_ref, ws_ref,
                            wb_ref, b_ref, bn2s_ref, bn2b_ref,
                            gf2_ref, gc2_ref, m=m)


def _head_kernel(p_ref, x_ref, wc_ref, bc_ref, wo_ref, bo_ref, o_ref):
    c = jnp.dot(p_ref[...], x_ref[...], preferred_element_type=jnp.float32)
    h = _softplus(c)
    h = jnp.dot(h, wc_ref[...], preferred_element_type=jnp.float32) + bc_ref[...]
    h = _softplus(h)
    o_ref[...] = (
        jnp.dot(h, wo_ref[...], preferred_element_type=jnp.float32) + bo_ref[...]
    )


def kernel(atom_fea, nbr_fea, nbr_fea_idx, pool_mat, emb_w, emb_b, fc_w, fc_b, out_w, out_b, conv0_wsf, conv0_wsc, conv0_wnf, conv0_wnc, conv0_wbf, conv0_wbc, conv0_bf, conv0_bc, conv0_bn2_s, conv0_bn2_b, conv1_wsf, conv1_wsc, conv1_wnf, conv1_wnc, conv1_wbf, conv1_wbc, conv1_bf, conv1_bc, conv1_bn2_s, conv1_bn2_b, conv2_wsf, conv2_wsc, conv2_wnf, conv2_wnc, conv2_wbf, conv2_wbc, conv2_bf, conv2_bc, conv2_bn2_s, conv2_bn2_b):
    N, M = nbr_fea_idx.shape
    B = nbr_fea.shape[2]
    F = atom_fea.shape[1]
    A = emb_w.shape[1]
    N0 = pool_mat.shape[0]

    T = 512 if N % 512 == 0 else N
    G = N // T

    cparams = pltpu.CompilerParams(dimension_semantics=("parallel",))

    convs = [
        (conv0_wsf, conv0_wsc, conv0_wnf, conv0_wnc, conv0_wbf, conv0_wbc,
         conv0_bf, conv0_bc, conv0_bn2_s, conv0_bn2_b),
        (conv1_wsf, conv1_wsc, conv1_wnf, conv1_wnc, conv1_wbf, conv1_wbc,
         conv1_bf, conv1_bc, conv1_bn2_s, conv1_bn2_b),
        (conv2_wsf, conv2_wsc, conv2_wnf, conv2_wnc, conv2_wbf, conv2_wbc,
         conv2_bf, conv2_bc, conv2_bn2_s, conv2_bn2_b),
    ]
    ws = [jnp.concatenate([c[0], c[1]], axis=1) for c in convs]
    wn = [jnp.concatenate([c[2], c[3]], axis=1) for c in convs]
    wb = [jnp.concatenate([c[4], c[5]], axis=1) for c in convs]
    bias = [jnp.concatenate([c[6], c[7]], axis=1) for c in convs]
    bn2s = [c[8] for c in convs]
    bn2b = [c[9] for c in convs]

    nb_flat = nbr_fea.reshape(N * M, B)

    const = lambda shape: pl.BlockSpec(shape, lambda i: (0, 0))

    x, pf, pc = pl.pallas_call(
        _embed_proj_kernel,
        out_shape=(jax.ShapeDtypeStruct((N, A), jnp.float32),
                   jax.ShapeDtypeStruct((N, A), jnp.float32),
                   jax.ShapeDtypeStruct((N, A), jnp.float32)),
        grid=(G,),
        in_specs=[pl.BlockSpec((T, F), lambda i: (i, 0)),
                  const((F, A)), const((1, A)), const((A, 2 * A))],
        out_specs=(pl.BlockSpec((T, A), lambda i: (i, 0)),
                   pl.BlockSpec((T, A), lambda i: (i, 0)),
                   pl.BlockSpec((T, A), lambda i: (i, 0))),
        compiler_params=cparams,
    )(atom_fea, emb_w, emb_b, wn[0])

    conv_in_specs = [
        pl.BlockSpec((T, A), lambda i: (i, 0)),
        pl.BlockSpec((N, 1, A), lambda i: (0, 0, 0)),
        pl.BlockSpec((N, 1, A), lambda i: (0, 0, 0)),
        pl.BlockSpec((T, M), lambda i: (i, 0),
                     memory_space=pltpu.MemorySpace.SMEM),
        pl.BlockSpec((T * M, B), lambda i: (i, 0)),
        const((A, 2 * A)), const((B, 2 * A)), const((1, 2 * A)),
        const((1, A)), const((1, A)),
    ]
    scratch = [pltpu.VMEM((T * M, A), jnp.float32),
               pltpu.VMEM((T * M, A), jnp.float32)]

    for layer in (0, 1):
        x, pf, pc = pl.pallas_call(
            functools.partial(_conv_proj_kernel, m=M),
            out_shape=(jax.ShapeDtypeStruct((N, A), jnp.float32),
                       jax.ShapeDtypeStruct((N, A), jnp.float32),
                       jax.ShapeDtypeStruct((N, A), jnp.float32)),
            grid=(G,),
            in_specs=conv_in_specs + [const((A, 2 * A))],
            out_specs=(pl.BlockSpec((T, A), lambda i: (i, 0)),
                       pl.BlockSpec((T, A), lambda i: (i, 0)),
                       pl.BlockSpec((T, A), lambda i: (i, 0))),
            scratch_shapes=scratch,
            compiler_params=cparams,
        )(x, pf.reshape(N, 1, A), pc.reshape(N, 1, A), nbr_fea_idx, nb_flat,
          ws[layer], wb[layer], bias[layer], bn2s[layer], bn2b[layer],
          wn[layer + 1])

    x = pl.pallas_call(
        functools.partial(_conv_last_kernel, m=M),
        out_shape=jax.ShapeDtypeStruct((N, A), jnp.float32),
        grid=(G,),
        in_specs=conv_in_specs,
        out_specs=pl.BlockSpec((T, A), lambda i: (i, 0)),
        scratch_shapes=scratch,
        compiler_params=cparams,
    )(x, pf.reshape(N, 1, A), pc.reshape(N, 1, A), nbr_fea_idx, nb_flat,
      ws[2], wb[2], bias[2], bn2s[2], bn2b[2])

    vspec = pl.BlockSpec(memory_space=pltpu.MemorySpace.VMEM)
    out = pl.pallas_call(
        _head_kernel,
        out_shape=jax.ShapeDtypeStruct((N0, 1), jnp.float32),
        in_specs=[vspec] * 6,
        out_specs=vspec,
    )(pool_mat, x, fc_w, fc_b, out_w, out_b)
    return out


# speedup vs baseline: 1.1040x; 1.0119x over previous
import functools

import jax
import jax.numpy as jnp
from jax.experimental import pallas as pl
from jax.experimental.pallas import tpu as pltpu


def _softplus(x):
    return jnp.maximum(x, 0.0) + jnp.log1p(jnp.exp(-jnp.abs(x)))


def _sigmoid(x):
    return 0.5 * (jnp.tanh(0.5 * x) + 1.0)


def _embed_proj_kernel(x_ref, w_ref, b_ref, wn_ref, o_ref, pf_ref, pc_ref):
    x = (
        jnp.dot(x_ref[...], w_ref[...], preferred_element_type=jnp.float32)
        + b_ref[...]
    )
    o_ref[...] = x
    A = x.shape[1]
    p = jnp.dot(x, wn_ref[...], preferred_element_type=jnp.float32)
    pf_ref[...] = p[:, :A]
    pc_ref[...] = p[:, A:]


def _conv_body(x_ref, pf3_ref, pc3_ref, idx_ref, nb_ref, ws_ref, wb_ref,
               b_ref, bn2s_ref, bn2b_ref, gf2_ref, gc2_ref, *, m):
    T, A = x_ref.shape

    U = 16
    E = U * m

    def gather_rows(tt, carry):
        base = tt * E
        idxs = [idx_ref[(tt * U) + u, mi]
                for u in range(U) for mi in range(m)]
        rowfs = [pf3_ref[i] for i in idxs]
        rowcs = [pc3_ref[i] for i in idxs]
        for k in range(E // 8):
            dst = pl.multiple_of(base + k * 8, 8)
            gf2_ref[pl.ds(dst, 8), :] = jnp.concatenate(
                rowfs[k * 8:(k + 1) * 8], axis=0)
            gc2_ref[pl.ds(dst, 8), :] = jnp.concatenate(
                rowcs[k * 8:(k + 1) * 8], axis=0)
        return carry

    jax.lax.fori_loop(0, T // U, gather_rows, 0)

    x = x_ref[...]
    zs = jnp.dot(x, ws_ref[...],
                 preferred_element_type=jnp.float32) + b_ref[...]
    zb = jnp.dot(nb_ref[...], wb_ref[...],
                 preferred_element_type=jnp.float32)
    zf = zb[:, :A] + gf2_ref[...]
    zc = zb[:, A:] + gc2_ref[...]
    zf = zf.reshape(T, m, A) + zs[:, None, :A]
    zc = zc.reshape(T, m, A) + zs[:, None, A:]
    gate = _sigmoid(zf) * _softplus(zc)
    summed = jnp.sum(gate, axis=1)
    return _softplus(x + summed * bn2s_ref[...] + bn2b_ref[...])


def _conv_proj_kernel(x_ref, pf3_ref, pc3_ref, idx_ref, nb_ref, ws_ref,
                      wb_ref, b_ref, bn2s_ref, bn2b_ref, wn_ref,
                      o_ref, pf_ref, pc_ref, gf2_ref, gc2_ref, *, m):
    y = _conv_body(x_ref, pf3_ref, pc3_ref, idx_ref, nb_ref, ws_ref, wb_ref,
                   b_ref, bn2s_ref, bn2b_ref, gf2_ref, gc2_ref, m=m)
    o_ref[...] = y
    A = y.shape[1]
    p = jnp.dot(y, wn_ref[...], preferred_element_type=jnp.float32)
    pf_ref[...] = p[:, :A]
    pc_ref[...] = p[:, A:]


def _conv_last_kernel(x_ref, pf3_ref, pc3_ref, idx_ref, nb_ref, ws_ref,
                      wb_ref, b_ref, bn2s_ref, bn2b_ref, o_ref,
                      gf2_ref, gc2_ref, *, m):
    o_ref[...] = _conv_body(x_ref, pf3_ref, pc3_ref, idx_ref, nb_ref, ws_ref,
                            wb_ref, b_ref, bn2s_ref, bn2b_ref,
                            gf2_ref, gc2_ref, m=m)


def _head_kernel(p_ref, x_ref, wc_ref, bc_ref, wo_ref, bo_ref, o_ref):
    c = jnp.dot(p_ref[...], x_ref[...], preferred_element_type=jnp.float32)
    h = _softplus(c)
    h = jnp.dot(h, wc_ref[...], preferred_element_type=jnp.float32) + bc_ref[...]
    h = _softplus(h)
    o_ref[...] = (
        jnp.dot(h, wo_ref[...], preferred_element_type=jnp.float32) + bo_ref[...]
    )


def kernel(atom_fea, nbr_fea, nbr_fea_idx, pool_mat, emb_w, emb_b, fc_w, fc_b, out_w, out_b, conv0_wsf, conv0_wsc, conv0_wnf, conv0_wnc, conv0_wbf, conv0_wbc, conv0_bf, conv0_bc, conv0_bn2_s, conv0_bn2_b, conv1_wsf, conv1_wsc, conv1_wnf, conv1_wnc, conv1_wbf, conv1_wbc, conv1_bf, conv1_bc, conv1_bn2_s, conv1_bn2_b, conv2_wsf, conv2_wsc, conv2_wnf, conv2_wnc, conv2_wbf, conv2_wbc, conv2_bf, conv2_bc, conv2_bn2_s, conv2_bn2_b):
    N, M = nbr_fea_idx.shape
    B = nbr_fea.shape[2]
    F = atom_fea.shape[1]
    A = emb_w.shape[1]
    N0 = pool_mat.shape[0]

    T = 512 if N % 512 == 0 else N
    G = N // T

    cparams = pltpu.CompilerParams(dimension_semantics=("parallel",))

    convs = [
        (conv0_wsf, conv0_wsc, conv0_wnf, conv0_wnc, conv0_wbf, conv0_wbc,
         conv0_bf, conv0_bc, conv0_bn2_s, conv0_bn2_b),
        (conv1_wsf, conv1_wsc, conv1_wnf, conv1_wnc, conv1_wbf, conv1_wbc,
         conv1_bf, conv1_bc, conv1_bn2_s, conv1_bn2_b),
        (conv2_wsf, conv2_wsc, conv2_wnf, conv2_wnc, conv2_wbf, conv2_wbc,
         conv2_bf, conv2_bc, conv2_bn2_s, conv2_bn2_b),
    ]
    ws = [jnp.concatenate([c[0], c[1]], axis=1) for c in convs]
    wn = [jnp.concatenate([c[2], c[3]], axis=1) for c in convs]
    wb = [jnp.concatenate([c[4], c[5]], axis=1) for c in convs]
    bias = [jnp.concatenate([c[6], c[7]], axis=1) for c in convs]
    bn2s = [c[8] for c in convs]
    bn2b = [c[9] for c in convs]

    nb_flat = nbr_fea.reshape(N * M, B)

    const = lambda shape: pl.BlockSpec(shape, lambda i: (0, 0))

    x, pf, pc = pl.pallas_call(
        _embed_proj_kernel,
        out_shape=(jax.ShapeDtypeStruct((N, A), jnp.float32),
                   jax.ShapeDtypeStruct((N, A), jnp.float32),
                   jax.ShapeDtypeStruct((N, A), jnp.float32)),
        grid=(G,),
        in_specs=[pl.BlockSpec((T, F), lambda i: (i, 0)),
                  const((F, A)), const((1, A)), const((A, 2 * A))],
        out_specs=(pl.BlockSpec((T, A), lambda i: (i, 0)),
                   pl.BlockSpec((T, A), lambda i: (i, 0)),
                   pl.BlockSpec((T, A), lambda i: (i, 0))),
        compiler_params=cparams,
    )(atom_fea, emb_w, emb_b, wn[0])

    conv_in_specs = [
        pl.BlockSpec((T, A), lambda i: (i, 0)),
        pl.BlockSpec((N, 1, A), lambda i: (0, 0, 0)),
        pl.BlockSpec((N, 1, A), lambda i: (0, 0, 0)),
        pl.BlockSpec((T, M), lambda i: (i, 0),
                     memory_space=pltpu.MemorySpace.SMEM),
        pl.BlockSpec((T * M, B), lambda i: (i, 0)),
        const((A, 2 * A)), const((B, 2 * A)), const((1, 2 * A)),
        const((1, A)), const((1, A)),
    ]
    scratch = [pltpu.VMEM((T * M, A), jnp.float32),
               pltpu.VMEM((T * M, A), jnp.float32)]

    for layer in (0, 1):
        x, pf, pc = pl.pallas_call(
            functools.partial(_conv_proj_kernel, m=M),
            out_shape=(jax.ShapeDtypeStruct((N, A), jnp.float32),
                       jax.ShapeDtypeStruct((N, A), jnp.float32),
                       jax.ShapeDtypeStruct((N, A), jnp.float32)),
            grid=(G,),
            in_specs=conv_in_specs + [const((A, 2 * A))],
            out_specs=(pl.BlockSpec((T, A), lambda i: (i, 0)),
                       pl.BlockSpec((T, A), lambda i: (i, 0)),
                       pl.BlockSpec((T, A), lambda i: (i, 0))),
            scratch_shapes=scratch,
            compiler_params=cparams,
        )(x, pf.reshape(N, 1, A), pc.reshape(N, 1, A), nbr_fea_idx, nb_flat,
          ws[layer], wb[layer], bias[layer], bn2s[layer], bn2b[layer],
          wn[layer + 1])

    x = pl.pallas_call(
        functools.partial(_conv_last_kernel, m=M),
        out_shape=jax.ShapeDtypeStruct((N, A), jnp.float32),
        grid=(G,),
        in_specs=conv_in_specs,
        out_specs=pl.BlockSpec((T, A), lambda i: (i, 0)),
        scratch_shapes=scratch,
        compiler_params=cparams,
    )(x, pf.reshape(N, 1, A), pc.reshape(N, 1, A), nbr_fea_idx, nb_flat,
      ws[2], wb[2], bias[2], bn2s[2], bn2b[2])

    vspec = pl.BlockSpec(memory_space=pltpu.MemorySpace.VMEM)
    out = pl.pallas_call(
        _head_kernel,
        out_shape=jax.ShapeDtypeStruct((N0, 1), jnp.float32),
        in_specs=[vspec] * 6,
        out_specs=vspec,
    )(pool_mat, x, fc_w, fc_b, out_w, out_b)
    return out


# speedup vs baseline: 1.1147x; 1.0096x over previous
import functools

import jax
import jax.numpy as jnp
from jax.experimental import pallas as pl
from jax.experimental.pallas import tpu as pltpu


def _softplus(x):
    return jnp.maximum(x, 0.0) + jnp.log1p(jnp.exp(-jnp.abs(x)))


def _sigmoid(x):
    return 0.5 * (jnp.tanh(0.5 * x) + 1.0)


def _embed_proj_kernel(x_ref, w_ref, b_ref, wn_ref, o_ref, pf_ref, pc_ref):
    x = (
        jnp.dot(x_ref[...], w_ref[...], preferred_element_type=jnp.float32)
        + b_ref[...]
    )
    o_ref[...] = x
    A = x.shape[1]
    p = jnp.dot(x, wn_ref[...], preferred_element_type=jnp.float32)
    pf_ref[...] = p[:, :A]
    pc_ref[...] = p[:, A:]


def _conv_body(x_ref, pf3_ref, pc3_ref, idx_ref, nb_ref, ws_ref, wb_ref,
               b_ref, bn2s_ref, bn2b_ref, gf2_ref, gc2_ref, *, m):
    T, A = x_ref.shape

    U = 32
    E = U * m

    def gather_rows(tt, carry):
        base = tt * E
        idxs = [idx_ref[(tt * U) + u, mi]
                for u in range(U) for mi in range(m)]
        rowfs = [pf3_ref[i] for i in idxs]
        rowcs = [pc3_ref[i] for i in idxs]
        for k in range(E // 8):
            dst = pl.multiple_of(base + k * 8, 8)
            gf2_ref[pl.ds(dst, 8), :] = jnp.concatenate(
                rowfs[k * 8:(k + 1) * 8], axis=0)
            gc2_ref[pl.ds(dst, 8), :] = jnp.concatenate(
                rowcs[k * 8:(k + 1) * 8], axis=0)
        return carry

    jax.lax.fori_loop(0, T // U, gather_rows, 0)

    x = x_ref[...]
    zs = jnp.dot(x, ws_ref[...],
                 preferred_element_type=jnp.float32) + b_ref[...]
    zb = jnp.dot(nb_ref[...], wb_ref[...],
                 preferred_element_type=jnp.float32)
    zf = zb[:, :A] + gf2_ref[...]
    zc = zb[:, A:] + gc2_ref[...]
    zf = zf.reshape(T, m, A) + zs[:, None, :A]
    zc = zc.reshape(T, m, A) + zs[:, None, A:]
    gate = _sigmoid(zf) * _softplus(zc)
    summed = jnp.sum(gate, axis=1)
    return _softplus(x + summed * bn2s_ref[...] + bn2b_ref[...])


def _conv_proj_kernel(x_ref, pf3_ref, pc3_ref, idx_ref, nb_ref, ws_ref,
                      wb_ref, b_ref, bn2s_ref, bn2b_ref, wn_ref,
                      o_ref, pf_ref, pc_ref, gf2_ref, gc2_ref, *, m):
    y = _conv_body(x_ref, pf3_ref, pc3_ref, idx_ref, nb_ref, ws_ref, wb_ref,
                   b_ref, bn2s_ref, bn2b_ref, gf2_ref, gc2_ref, m=m)
    o_ref[...] = y
    A = y.shape[1]
    p = jnp.dot(y, wn_ref[...], preferred_element_type=jnp.float32)
    pf_ref[...] = p[:, :A]
    pc_ref[...] = p[:, A:]


def _conv_last_kernel(x_ref, pf3_ref, pc3_ref, idx_ref, nb_ref, ws_ref,
                      wb_ref, b_ref, bn2s_ref, bn2b_ref, o_ref,
                      gf2_ref, gc2_ref, *, m):
    o_ref[...] = _conv_body(x_ref, pf3_ref, pc3_ref, idx_ref, nb_ref, ws_ref,
                            wb_ref, b_ref, bn2s_ref, bn2b_ref,
                            gf2_ref, gc2_ref, m=m)


def _head_kernel(p_ref, x_ref, wc_ref, bc_ref, wo_ref, bo_ref, o_ref):
    c = jnp.dot(p_ref[...], x_ref[...], preferred_element_type=jnp.float32)
    h = _softplus(c)
    h = jnp.dot(h, wc_ref[...], preferred_element_type=jnp.float32) + bc_ref[...]
    h = _softplus(h)
    o_ref[...] = (
        jnp.dot(h, wo_ref[...], preferred_element_type=jnp.float32) + bo_ref[...]
    )


def kernel(atom_fea, nbr_fea, nbr_fea_idx, pool_mat, emb_w, emb_b, fc_w, fc_b, out_w, out_b, conv0_wsf, conv0_wsc, conv0_wnf, conv0_wnc, conv0_wbf, conv0_wbc, conv0_bf, conv0_bc, conv0_bn2_s, conv0_bn2_b, conv1_wsf, conv1_wsc, conv1_wnf, conv1_wnc, conv1_wbf, conv1_wbc, conv1_bf, conv1_bc, conv1_bn2_s, conv1_bn2_b, conv2_wsf, conv2_wsc, conv2_wnf, conv2_wnc, conv2_wbf, conv2_wbc, conv2_bf, conv2_bc, conv2_bn2_s, conv2_bn2_b):
    N, M = nbr_fea_idx.shape
    B = nbr_fea.shape[2]
    F = atom_fea.shape[1]
    A = emb_w.shape[1]
    N0 = pool_mat.shape[0]

    T = 512 if N % 512 == 0 else N
    G = N // T

    cparams = pltpu.CompilerParams(dimension_semantics=("parallel",))

    convs = [
        (conv0_wsf, conv0_wsc, conv0_wnf, conv0_wnc, conv0_wbf, conv0_wbc,
         conv0_bf, conv0_bc, conv0_bn2_s, conv0_bn2_b),
        (conv1_wsf, conv1_wsc, conv1_wnf, conv1_wnc, conv1_wbf, conv1_wbc,
         conv1_bf, conv1_bc, conv1_bn2_s, conv1_bn2_b),
        (conv2_wsf, conv2_wsc, conv2_wnf, conv2_wnc, conv2_wbf, conv2_wbc,
         conv2_bf, conv2_bc, conv2_bn2_s, conv2_bn2_b),
    ]
    ws = [jnp.concatenate([c[0], c[1]], axis=1) for c in convs]
    wn = [jnp.concatenate([c[2], c[3]], axis=1) for c in convs]
    wb = [jnp.concatenate([c[4], c[5]], axis=1) for c in convs]
    bias = [jnp.concatenate([c[6], c[7]], axis=1) for c in convs]
    bn2s = [c[8] for c in convs]
    bn2b = [c[9] for c in convs]

    nb_flat = nbr_fea.reshape(N * M, B)

    const = lambda shape: pl.BlockSpec(shape, lambda i: (0, 0))

    x, pf, pc = pl.pallas_call(
        _embed_proj_kernel,
        out_shape=(jax.ShapeDtypeStruct((N, A), jnp.float32),
                   jax.ShapeDtypeStruct((N, A), jnp.float32),
                   jax.ShapeDtypeStruct((N, A), jnp.float32)),
        grid=(G,),
        in_specs=[pl.BlockSpec((T, F), lambda i: (i, 0)),
                  const((F, A)), const((1, A)), const((A, 2 * A))],
        out_specs=(pl.BlockSpec((T, A), lambda i: (i, 0)),
                   pl.BlockSpec((T, A), lambda i: (i, 0)),
                   pl.BlockSpec((T, A), lambda i: (i, 0))),
        compiler_params=cparams,
    )(atom_fea, emb_w, emb_b, wn[0])

    conv_in_specs = [
        pl.BlockSpec((T, A), lambda i: (i, 0)),
        pl.BlockSpec((N, 1, A), lambda i: (0, 0, 0)),
        pl.BlockSpec((N, 1, A), lambda i: (0, 0, 0)),
        pl.BlockSpec((T, M), lambda i: (i, 0),
                     memory_space=pltpu.MemorySpace.SMEM),
        pl.BlockSpec((T * M, B), lambda i: (i, 0)),
        const((A, 2 * A)), const((B, 2 * A)), const((1, 2 * A)),
        const((1, A)), const((1, A)),
    ]
    scratch = [pltpu.VMEM((T * M, A), jnp.float32),
               pltpu.VMEM((T * M, A), jnp.float32)]

    for layer in (0, 1):
        x, pf, pc = pl.pallas_call(
            functools.partial(_conv_proj_kernel, m=M),
            out_shape=(jax.ShapeDtypeStruct((N, A), jnp.float32),
                       jax.ShapeDtypeStruct((N, A), jnp.float32),
                       jax.ShapeDtypeStruct((N, A), jnp.float32)),
            grid=(G,),
            in_specs=conv_in_specs + [const((A, 2 * A))],
            out_specs=(pl.BlockSpec((T, A), lambda i: (i, 0)),
                       pl.BlockSpec((T, A), lambda i: (i, 0)),
                       pl.BlockSpec((T, A), lambda i: (i, 0))),
            scratch_shapes=scratch,
            compiler_params=cparams,
        )(x, pf.reshape(N, 1, A), pc.reshape(N, 1, A), nbr_fea_idx, nb_flat,
          ws[layer], wb[layer], bias[layer], bn2s[layer], bn2b[layer],
          wn[layer + 1])

    x = pl.pallas_call(
        functools.partial(_conv_last_kernel, m=M),
        out_shape=jax.ShapeDtypeStruct((N, A), jnp.float32),
        grid=(G,),
        in_specs=conv_in_specs,
        out_specs=pl.BlockSpec((T, A), lambda i: (i, 0)),
        scratch_shapes=scratch,
        compiler_params=cparams,
    )(x, pf.reshape(N, 1, A), pc.reshape(N, 1, A), nbr_fea_idx, nb_flat,
      ws[2], wb[2], bias[2], bn2s[2], bn2b[2])

    vspec = pl.BlockSpec(memory_space=pltpu.MemorySpace.VMEM)
    out = pl.pallas_call(
        _head_kernel,
        out_shape=jax.ShapeDtypeStruct((N0, 1), jnp.float32),
        in_specs=[vspec] * 6,
        out_specs=vspec,
    )(pool_mat, x, fc_w, fc_b, out_w, out_b)
    return out


# speedup vs baseline: 1.1883x; 1.0661x over previous
import functools

import jax
import jax.numpy as jnp
from jax.experimental import pallas as pl
from jax.experimental.pallas import tpu as pltpu


def _softplus(x):
    a = jnp.abs(x)
    return 0.5 * (x + a) + jnp.log(1.0 + jnp.exp(-a))


def _sigmoid(x):
    return 0.5 * (jnp.tanh(0.5 * x) + 1.0)


def _embed_proj_kernel(x_ref, w_ref, b_ref, wn_ref, o_ref, pf_ref, pc_ref):
    x = (
        jnp.dot(x_ref[...], w_ref[...], preferred_element_type=jnp.float32)
        + b_ref[...]
    )
    o_ref[...] = x
    A = x.shape[1]
    p = jnp.dot(x, wn_ref[...], preferred_element_type=jnp.float32)
    pf_ref[...] = p[:, :A]
    pc_ref[...] = p[:, A:]


def _conv_body(x_ref, pf3_ref, pc3_ref, idx_ref, nb_ref, ws_ref, wb_ref,
               b_ref, bn2s_ref, bn2b_ref, gf2_ref, gc2_ref, *, m):
    T, A = x_ref.shape

    U = 32
    E = U * m

    def gather_rows(tt, carry):
        base = tt * E
        idxs = [idx_ref[(tt * U) + u, mi]
                for u in range(U) for mi in range(m)]
        rowfs = [pf3_ref[i] for i in idxs]
        rowcs = [pc3_ref[i] for i in idxs]
        for k in range(E // 8):
            dst = pl.multiple_of(base + k * 8, 8)
            gf2_ref[pl.ds(dst, 8), :] = jnp.concatenate(
                rowfs[k * 8:(k + 1) * 8], axis=0)
            gc2_ref[pl.ds(dst, 8), :] = jnp.concatenate(
                rowcs[k * 8:(k + 1) * 8], axis=0)
        return carry

    jax.lax.fori_loop(0, T // U, gather_rows, 0)

    x = x_ref[...]
    zs = jnp.dot(x, ws_ref[...],
                 preferred_element_type=jnp.float32) + b_ref[...]
    zb = jnp.dot(nb_ref[...], wb_ref[...],
                 preferred_element_type=jnp.float32)
    zf = zb[:, :A] + gf2_ref[...]
    zc = zb[:, A:] + gc2_ref[...]
    zf = zf.reshape(T, m, A) + zs[:, None, :A]
    zc = zc.reshape(T, m, A) + zs[:, None, A:]
    gate = _sigmoid(zf) * _softplus(zc)
    summed = jnp.sum(gate, axis=1)
    return _softplus(x + summed * bn2s_ref[...] + bn2b_ref[...])


def _conv_proj_kernel(x_ref, pf3_ref, pc3_ref, idx_ref, nb_ref, ws_ref,
                      wb_ref, b_ref, bn2s_ref, bn2b_ref, wn_ref,
                      o_ref, pf_ref, pc_ref, gf2_ref, gc2_ref, *, m):
    y = _conv_body(x_ref, pf3_ref, pc3_ref, idx_ref, nb_ref, ws_ref, wb_ref,
                   b_ref, bn2s_ref, bn2b_ref, gf2_ref, gc2_ref, m=m)
    o_ref[...] = y
    A = y.shape[1]
    p = jnp.dot(y, wn_ref[...], preferred_element_type=jnp.float32)
    pf_ref[...] = p[:, :A]
    pc_ref[...] = p[:, A:]


def _conv_last_kernel(x_ref, pf3_ref, pc3_ref, idx_ref, nb_ref, ws_ref,
                      wb_ref, b_ref, bn2s_ref, bn2b_ref, o_ref,
                      gf2_ref, gc2_ref, *, m):
    o_ref[...] = _conv_body(x_ref, pf3_ref, pc3_ref, idx_ref, nb_ref, ws_ref,
                            wb_ref, b_ref, bn2s_ref, bn2b_ref,
                            gf2_ref, gc2_ref, m=m)


def _head_kernel(p_ref, x_ref, wc_ref, bc_ref, wo_ref, bo_ref, o_ref):
    c = jnp.dot(p_ref[...], x_ref[...], preferred_element_type=jnp.float32)
    h = _softplus(c)
    h = jnp.dot(h, wc_ref[...], preferred_element_type=jnp.float32) + bc_ref[...]
    h = _softplus(h)
    o_ref[...] = (
        jnp.dot(h, wo_ref[...], preferred_element_type=jnp.float32) + bo_ref[...]
    )


def kernel(atom_fea, nbr_fea, nbr_fea_idx, pool_mat, emb_w, emb_b, fc_w, fc_b, out_w, out_b, conv0_wsf, conv0_wsc, conv0_wnf, conv0_wnc, conv0_wbf, conv0_wbc, conv0_bf, conv0_bc, conv0_bn2_s, conv0_bn2_b, conv1_wsf, conv1_wsc, conv1_wnf, conv1_wnc, conv1_wbf, conv1_wbc, conv1_bf, conv1_bc, conv1_bn2_s, conv1_bn2_b, conv2_wsf, conv2_wsc, conv2_wnf, conv2_wnc, conv2_wbf, conv2_wbc, conv2_bf, conv2_bc, conv2_bn2_s, conv2_bn2_b):
    N, M = nbr_fea_idx.shape
    B = nbr_fea.shape[2]
    F = atom_fea.shape[1]
    A = emb_w.shape[1]
    N0 = pool_mat.shape[0]

    T = 512 if N % 512 == 0 else N
    G = N // T

    cparams = pltpu.CompilerParams(dimension_semantics=("parallel",))

    convs = [
        (conv0_wsf, conv0_wsc, conv0_wnf, conv0_wnc, conv0_wbf, conv0_wbc,
         conv0_bf, conv0_bc, conv0_bn2_s, conv0_bn2_b),
        (conv1_wsf, conv1_wsc, conv1_wnf, conv1_wnc, conv1_wbf, conv1_wbc,
         conv1_bf, conv1_bc, conv1_bn2_s, conv1_bn2_b),
        (conv2_wsf, conv2_wsc, conv2_wnf, conv2_wnc, conv2_wbf, conv2_wbc,
         conv2_bf, conv2_bc, conv2_bn2_s, conv2_bn2_b),
    ]
    ws = [jnp.concatenate([c[0], c[1]], axis=1) for c in convs]
    wn = [jnp.concatenate([c[2], c[3]], axis=1) for c in convs]
    wb = [jnp.concatenate([c[4], c[5]], axis=1) for c in convs]
    bias = [jnp.concatenate([c[6], c[7]], axis=1) for c in convs]
    bn2s = [c[8] for c in convs]
    bn2b = [c[9] for c in convs]

    nb_flat = nbr_fea.reshape(N * M, B)

    const = lambda shape: pl.BlockSpec(shape, lambda i: (0, 0))

    x, pf, pc = pl.pallas_call(
        _embed_proj_kernel,
        out_shape=(jax.ShapeDtypeStruct((N, A), jnp.float32),
                   jax.ShapeDtypeStruct((N, A), jnp.float32),
                   jax.ShapeDtypeStruct((N, A), jnp.float32)),
        grid=(G,),
        in_specs=[pl.BlockSpec((T, F), lambda i: (i, 0)),
                  const((F, A)), const((1, A)), const((A, 2 * A))],
        out_specs=(pl.BlockSpec((T, A), lambda i: (i, 0)),
                   pl.BlockSpec((T, A), lambda i: (i, 0)),
                   pl.BlockSpec((T, A), lambda i: (i, 0))),
        compiler_params=cparams,
    )(atom_fea, emb_w, emb_b, wn[0])

    conv_in_specs = [
        pl.BlockSpec((T, A), lambda i: (i, 0)),
        pl.BlockSpec((N, 1, A), lambda i: (0, 0, 0)),
        pl.BlockSpec((N, 1, A), lambda i: (0, 0, 0)),
        pl.BlockSpec((T, M), lambda i: (i, 0),
                     memory_space=pltpu.MemorySpace.SMEM),
        pl.BlockSpec((T * M, B), lambda i: (i, 0)),
        const((A, 2 * A)), const((B, 2 * A)), const((1, 2 * A)),
        const((1, A)), const((1, A)),
    ]
    scratch = [pltpu.VMEM((T * M, A), jnp.float32),
               pltpu.VMEM((T * M, A), jnp.float32)]

    for layer in (0, 1):
        x, pf, pc = pl.pallas_call(
            functools.partial(_conv_proj_kernel, m=M),
            out_shape=(jax.ShapeDtypeStruct((N, A), jnp.float32),
                       jax.ShapeDtypeStruct((N, A), jnp.float32),
                       jax.ShapeDtypeStruct((N, A), jnp.float32)),
            grid=(G,),
            in_specs=conv_in_specs + [const((A, 2 * A))],
            out_specs=(pl.BlockSpec((T, A), lambda i: (i, 0)),
                       pl.BlockSpec((T, A), lambda i: (i, 0)),
                       pl.BlockSpec((T, A), lambda i: (i, 0))),
            scratch_shapes=scratch,
            compiler_params=cparams,
        )(x, pf.reshape(N, 1, A), pc.reshape(N, 1, A), nbr_fea_idx, nb_flat,
          ws[layer], wb[layer], bias[layer], bn2s[layer], bn2b[layer],
          wn[layer + 1])

    x = pl.pallas_call(
        functools.partial(_conv_last_kernel, m=M),
        out_shape=jax.ShapeDtypeStruct((N, A), jnp.float32),
        grid=(G,),
        in_specs=conv_in_specs,
        out_specs=pl.BlockSpec((T, A), lambda i: (i, 0)),
        scratch_shapes=scratch,
        compiler_params=cparams,
    )(x, pf.reshape(N, 1, A), pc.reshape(N, 1, A), nbr_fea_idx, nb_flat,
      ws[2], wb[2], bias[2], bn2s[2], bn2b[2])

    vspec = pl.BlockSpec(memory_space=pltpu.MemorySpace.VMEM)
    out = pl.pallas_call(
        _head_kernel,
        out_shape=jax.ShapeDtypeStruct((N0, 1), jnp.float32),
        in_specs=[vspec] * 6,
        out_specs=vspec,
    )(pool_mat, x, fc_w, fc_b, out_w, out_b)
    return out
